# Initial kernel scaffold; baseline (speedup 1.0000x reference)
import functools
import jax
import jax.numpy as jnp
from jax.experimental import pallas as pl
from jax.experimental.pallas import tpu as pltpu

NUM_Q = 20000
NUM_C = 80
NUM_GT = 100
LANES = 128
QBLK = 1000
NBLK = NUM_Q // QBLK
COST_CLASS = 2.0
COST_BBOX = 5.0
COST_GIOU = 2.0
ALPHA = 0.25
OTA_K = 5
BIG = 3.0e38


def _matcher_kernel(logits_ref, boxes_ref, tgtT_ref, labels_ref,
                    cost_ref, fg_ref, mgt_ref, qidx_ref,
                    t5i_ref, t5c_ref):
    i = pl.program_id(0)
    lane = jax.lax.broadcasted_iota(jnp.int32, (1, LANES), 1)
    gmask = lane < NUM_GT

    @pl.when(i == 0)
    def _init():
        t5i_ref[...] = jnp.full((8, LANES), -BIG, jnp.float32)
        t5c_ref[...] = jnp.full((8, LANES), BIG, jnp.float32)

    g = [tgtT_ref[k:k + 1, :] for k in range(8)]
    gxc = (g[0] + g[2] + g[4] + g[6]) / 4.0
    gyc = (g[1] + g[3] + g[5] + g[7]) / 4.0
    gw = jnp.sqrt((g[0] - g[2]) ** 2 + (g[1] - g[3]) ** 2)
    gh = jnp.sqrt((g[2] - g[4]) ** 2 + (g[3] - g[5]) ** 2)
    bx1 = gxc - gw / 2.0
    by1 = gyc - gh / 2.0
    bx2 = gxc + gw / 2.0
    by2 = gyc + gh / 2.0
    area_b = (bx2 - bx1) * (by2 - by1)
    v1x = g[2] - g[0]; v1y = g[3] - g[1]
    v2x = g[4] - g[0]; v2y = g[5] - g[1]
    gt_area = jnp.abs(v1x * v2y - v1y * v2x) / 2.0
    pxmax = jnp.maximum(jnp.maximum(g[0], g[2]), jnp.maximum(g[4], g[6]))
    pxmin = jnp.minimum(jnp.minimum(g[0], g[2]), jnp.minimum(g[4], g[6]))
    pymax = jnp.maximum(jnp.maximum(g[1], g[3]), jnp.maximum(g[5], g[7]))
    pymin = jnp.minimum(jnp.minimum(g[1], g[3]), jnp.minimum(g[5], g[7]))
    max_diag = jnp.sqrt((pxmax - pxmin) ** 2 + (pymax - pymin) ** 2)
    radii = max_diag / 32.0
    thr_a = gt_area / (radii + 1e-12)

    c = [boxes_ref[:, k:k + 1] for k in range(8)]
    qxc = (c[0] + c[2] + c[4] + c[6]) / 4.0
    qyc = (c[1] + c[3] + c[5] + c[7]) / 4.0
    qw = jnp.sqrt((c[0] - c[2]) ** 2 + (c[1] - c[3]) ** 2)
    qh = jnp.sqrt((c[2] - c[4]) ** 2 + (c[3] - c[5]) ** 2)
    ax1 = qxc - qw / 2.0
    ay1 = qyc - qh / 2.0
    ax2 = qxc + qw / 2.0
    ay2 = qyc + qh / 2.0
    area_a = (ax2 - ax1) * (ay2 - ay1)

    lg = logits_ref[...]
    p = jax.nn.sigmoid(lg)
    neg = (1.0 - ALPHA) * (p * p) * (-jnp.log(1.0 - p + 1e-8))
    pos = ALPHA * ((1.0 - p) * (1.0 - p)) * (-jnp.log(p + 1e-8))
    cc80 = pos - neg
    cls = jax.lax.broadcasted_iota(jnp.int32, (NUM_C, LANES), 0)
    onehot = (cls == labels_ref[...]).astype(jnp.float32)
    cost_class = jax.lax.dot_general(
        cc80, onehot, (((1,), (0,)), ((), ())),
        preferred_element_type=jnp.float32)

    d2 = (qxc - gxc) ** 2 + (qyc - gyc) ** 2
    cd = jnp.sqrt(d2)
    in_any = (cd <= thr_a) | (cd <= radii)
    fg_row = jnp.max(jnp.where(gmask & in_any, 1, 0), axis=1, keepdims=True) > 0

    iw = jnp.maximum(jnp.minimum(ax2, bx2) - jnp.maximum(ax1, bx1), 0.0)
    ih = jnp.maximum(jnp.minimum(ay2, by2) - jnp.maximum(ay1, by1), 0.0)
    inter = iw * ih
    union = area_a + area_b - inter
    iou = inter / (union + 1e-8)

    ex1 = jnp.minimum(ax1, bx1)
    ey1 = jnp.minimum(ay1, by1)
    ex2 = jnp.maximum(ax2, bx2)
    ey2 = jnp.maximum(ay2, by2)
    diag2 = (ex2 - ex1) ** 2 + (ey2 - ey1) ** 2
    diou = iou - d2 / (diag2 + 1e-8)

    cb = (jnp.abs(c[0] - g[0]) + jnp.abs(c[1] - g[1]) + jnp.abs(c[2] - g[2])
          + jnp.abs(c[3] - g[3]) + jnp.abs(c[4] - g[4]) + jnp.abs(c[5] - g[5])
          + jnp.abs(c[6] - g[6]) + jnp.abs(c[7] - g[7]))

    cost = COST_BBOX * cb + COST_CLASS * cost_class + COST_GIOU * diou
    cost = jnp.where(fg_row, cost, cost + 10000.0)

    cost_ref[pl.ds(i * QBLK, QBLK), :] = cost[:, :NUM_GT]

    rio = jax.lax.broadcasted_iota(jnp.int32, (QBLK + 8, 1), 0)

    iu = jnp.where(gmask, iou, -BIG)
    work = jnp.concatenate([iu, t5i_ref[...]], axis=0)
    rows = []
    for _ in range(OTA_K):
        m = jnp.max(work, axis=0, keepdims=True)
        ii = jnp.min(jnp.where(work == m, rio, QBLK + 8), axis=0, keepdims=True)
        work = jnp.where(rio == ii, -BIG, work)
        rows.append(m)
    rows += [jnp.full((1, LANES), -BIG, jnp.float32)] * 3
    t5i_ref[...] = jnp.concatenate(rows, axis=0)

    cu = jnp.where(gmask, cost, BIG)
    work = jnp.concatenate([cu, t5c_ref[...]], axis=0)
    rows = []
    for _ in range(OTA_K):
        m = jnp.min(work, axis=0, keepdims=True)
        ii = jnp.min(jnp.where(work == m, rio, QBLK + 8), axis=0, keepdims=True)
        work = jnp.where(rio == ii, BIG, work)
        rows.append(m)
    rows += [jnp.full((1, LANES), BIG, jnp.float32)] * 3
    t5c_ref[...] = jnp.concatenate(rows, axis=0)

    @pl.when(i == NBLK - 1)
    def _finish():
        s = (t5i_ref[0:1, :] + t5i_ref[1:2, :] + t5i_ref[2:3, :]
             + t5i_ref[3:4, :] + t5i_ref[4:5, :])
        dkm1 = jnp.clip(s.astype(jnp.int32), 1, None) - 1
        kth = jnp.zeros((1, LANES), jnp.float32)
        for r in range(OTA_K):
            kth = kth + jnp.where(dkm1 == r, t5c_ref[r:r + 1, :], 0.0)
        kth100 = kth[:, :NUM_GT]

        lane100 = jax.lax.broadcasted_iota(jnp.int32, (1, NUM_GT), 1)
        rio_q = jax.lax.broadcasted_iota(jnp.int32, (QBLK, 1), 0)
        run_val = jnp.full((1, NUM_GT), BIG, jnp.float32)
        run_idx = jnp.zeros((1, NUM_GT), jnp.int32)
        for k in range(NBLK):
            cblk = cost_ref[k * QBLK:(k + 1) * QBLK, :]
            match = cblk <= kth100
            cnt = jnp.sum(match.astype(jnp.int32), axis=1, keepdims=True)
            fj = jnp.min(jnp.where(match, lane100, NUM_GT), axis=1,
                         keepdims=True)
            cmin = jnp.min(cblk, axis=1, keepdims=True)
            mg = jnp.min(jnp.where(cblk == cmin, lane100, NUM_GT), axis=1,
                         keepdims=True)
            mgt = jnp.where(cnt > 1, mg, jnp.where(cnt == 1, fj, -1))
            fg_ref[k * QBLK:(k + 1) * QBLK, :] = (cnt > 0).astype(jnp.int32)
            mgt_ref[k * QBLK:(k + 1) * QBLK, :] = mgt
            bmin = jnp.min(cblk, axis=0, keepdims=True)
            bidx = jnp.min(jnp.where(cblk == bmin, rio_q + k * QBLK, NUM_Q),
                           axis=0, keepdims=True)
            upd = bmin < run_val
            run_idx = jnp.where(upd, bidx, run_idx)
            run_val = jnp.where(upd, bmin, run_val)
        qidx_ref[...] = run_idx


@jax.jit
def kernel(pred_logits, pred_boxes, tgt_boxes, tgt_labels):
    logits = pred_logits[0]
    boxes = pred_boxes[0]
    tgtT = jnp.zeros((8, LANES), jnp.float32).at[:, :NUM_GT].set(tgt_boxes.T)
    labels_row = jnp.zeros((1, LANES), jnp.int32).at[0, :NUM_GT].set(
        tgt_labels.astype(jnp.int32))

    cost, fg, mgt, qidx = pl.pallas_call(
        _matcher_kernel,
        grid=(NBLK,),
        in_specs=[
            pl.BlockSpec((QBLK, NUM_C), lambda i: (i, 0)),
            pl.BlockSpec((QBLK, 8), lambda i: (i, 0)),
            pl.BlockSpec((8, LANES), lambda i: (0, 0)),
            pl.BlockSpec((1, LANES), lambda i: (0, 0)),
        ],
        out_specs=[
            pl.BlockSpec((NUM_Q, NUM_GT), lambda i: (0, 0)),
            pl.BlockSpec((NUM_Q, 1), lambda i: (0, 0)),
            pl.BlockSpec((NUM_Q, 1), lambda i: (0, 0)),
            pl.BlockSpec((1, NUM_GT), lambda i: (0, 0)),
        ],
        out_shape=[
            jax.ShapeDtypeStruct((NUM_Q, NUM_GT), jnp.float32),
            jax.ShapeDtypeStruct((NUM_Q, 1), jnp.int32),
            jax.ShapeDtypeStruct((NUM_Q, 1), jnp.int32),
            jax.ShapeDtypeStruct((1, NUM_GT), jnp.int32),
        ],
        scratch_shapes=[
            pltpu.VMEM((8, LANES), jnp.float32),
            pltpu.VMEM((8, LANES), jnp.float32),
        ],
        compiler_params=pltpu.CompilerParams(
            dimension_semantics=("arbitrary",)),
    )(logits, boxes, tgtT, labels_row)

    fg_match = fg.reshape(NUM_Q) > 0
    matched_gt = mgt.reshape(NUM_Q)
    matched_qidx = qidx.reshape(NUM_GT)
    return (fg_match, matched_gt, matched_qidx, cost)


# speedup vs baseline: 4.8910x; 4.8910x over previous
import jax
import jax.numpy as jnp
from jax.experimental import pallas as pl
from jax.experimental.pallas import tpu as pltpu

NUM_Q = 20000
NUM_C = 80
NUM_GT = 100
LANES = 128
QBLK = 400
NBLK = NUM_Q // QBLK
COST_CLASS = 2.0
COST_BBOX = 5.0
COST_GIOU = 2.0
ALPHA = 0.25
OTA_K = 5
BIG = 3.0e38


def _matcher_kernel(logits_ref, logp_ref, log1_ref, boxes_ref, tgtT_ref,
                    labels_ref, cost_ref, mgt_ref, qidx_ref,
                    t5i_ref, t5c_ref):
    i = pl.program_id(0)
    lane = jax.lax.broadcasted_iota(jnp.int32, (1, LANES), 1)
    gmask = lane < NUM_GT

    @pl.when(i == 0)
    def _init():
        t5i_ref[...] = jnp.full((8, LANES), -BIG, jnp.float32)
        t5c_ref[...] = jnp.full((8, LANES), BIG, jnp.float32)

    g = [tgtT_ref[k:k + 1, :] for k in range(8)]
    gxc = (g[0] + g[2] + g[4] + g[6]) / 4.0
    gyc = (g[1] + g[3] + g[5] + g[7]) / 4.0
    gw = jnp.sqrt((g[0] - g[2]) ** 2 + (g[1] - g[3]) ** 2)
    gh = jnp.sqrt((g[2] - g[4]) ** 2 + (g[3] - g[5]) ** 2)
    bx1 = gxc - gw / 2.0
    by1 = gyc - gh / 2.0
    bx2 = gxc + gw / 2.0
    by2 = gyc + gh / 2.0
    area_b = (bx2 - bx1) * (by2 - by1)
    v1x = g[2] - g[0]; v1y = g[3] - g[1]
    v2x = g[4] - g[0]; v2y = g[5] - g[1]
    gt_area = jnp.abs(v1x * v2y - v1y * v2x) / 2.0
    pxmax = jnp.maximum(jnp.maximum(g[0], g[2]), jnp.maximum(g[4], g[6]))
    pxmin = jnp.minimum(jnp.minimum(g[0], g[2]), jnp.minimum(g[4], g[6]))
    pymax = jnp.maximum(jnp.maximum(g[1], g[3]), jnp.maximum(g[5], g[7]))
    pymin = jnp.minimum(jnp.minimum(g[1], g[3]), jnp.minimum(g[5], g[7]))
    max_diag = jnp.sqrt((pxmax - pxmin) ** 2 + (pymax - pymin) ** 2)
    radii = max_diag / 32.0
    thr_a = gt_area / (radii + 1e-12)

    c = [boxes_ref[:, k:k + 1] for k in range(8)]
    qxc = (c[0] + c[2] + c[4] + c[6]) / 4.0
    qyc = (c[1] + c[3] + c[5] + c[7]) / 4.0
    qw = jnp.sqrt((c[0] - c[2]) ** 2 + (c[1] - c[3]) ** 2)
    qh = jnp.sqrt((c[2] - c[4]) ** 2 + (c[3] - c[5]) ** 2)
    ax1 = qxc - qw / 2.0
    ay1 = qyc - qh / 2.0
    ax2 = qxc + qw / 2.0
    ay2 = qyc + qh / 2.0
    area_a = (ax2 - ax1) * (ay2 - ay1)

    lg = logits_ref[...]
    p = jax.nn.sigmoid(lg)
    neg = (1 - ALPHA) * p ** 2.0 * (-log1_ref[...])
    pos = ALPHA * (1 - p) ** 2.0 * (-logp_ref[...])
    cc80 = pos - neg
    lbl = labels_ref[...]
    cost_class = jnp.zeros((QBLK, LANES), jnp.float32)
    for cidx in range(NUM_C):
        cost_class = jnp.where(lbl == cidx, cc80[:, cidx:cidx + 1],
                               cost_class)

    d2 = (qxc - gxc) ** 2 + (qyc - gyc) ** 2
    cd = jnp.sqrt(d2)
    in_any = (cd <= thr_a) | (cd <= radii)
    fg_row = jnp.max(jnp.where(gmask & in_any, 1, 0), axis=1, keepdims=True) > 0

    iw = jnp.maximum(jnp.minimum(ax2, bx2) - jnp.maximum(ax1, bx1), 0.0)
    ih = jnp.maximum(jnp.minimum(ay2, by2) - jnp.maximum(ay1, by1), 0.0)
    inter = iw * ih
    union = area_a + area_b - inter
    iou = inter / (union + 1e-8)

    ex1 = jnp.minimum(ax1, bx1)
    ey1 = jnp.minimum(ay1, by1)
    ex2 = jnp.maximum(ax2, bx2)
    ey2 = jnp.maximum(ay2, by2)
    diag2 = (ex2 - ex1) ** 2 + (ey2 - ey1) ** 2
    diou = iou - d2 / (diag2 + 1e-8)

    ab = [jnp.abs(c[k] - g[k]) for k in range(8)]
    cb = (((ab[0] + ab[4]) + (ab[2] + ab[6]))
          + ((ab[1] + ab[5]) + (ab[3] + ab[7])))

    cost = COST_BBOX * cb + COST_CLASS * cost_class + COST_GIOU * diou
    cost = jnp.where(fg_row, cost, cost + 10000.0)

    cost_ref[pl.ds(i * QBLK, QBLK), :] = cost[:, :NUM_GT]

    rio = jax.lax.broadcasted_iota(jnp.int32, (QBLK + 8, 1), 0)

    iu = jnp.where(gmask, iou, -BIG)
    work = jnp.concatenate([iu, t5i_ref[...]], axis=0)
    rows = []
    for _ in range(OTA_K):
        m = jnp.max(work, axis=0, keepdims=True)
        ii = jnp.min(jnp.where(work == m, rio, QBLK + 8), axis=0, keepdims=True)
        work = jnp.where(rio == ii, -BIG, work)
        rows.append(m)
    rows += [jnp.full((1, LANES), -BIG, jnp.float32)] * 3
    t5i_ref[...] = jnp.concatenate(rows, axis=0)

    cu = jnp.where(gmask, cost, BIG)
    work = jnp.concatenate([cu, t5c_ref[...]], axis=0)
    rows = []
    for _ in range(OTA_K):
        m = jnp.min(work, axis=0, keepdims=True)
        ii = jnp.min(jnp.where(work == m, rio, QBLK + 8), axis=0, keepdims=True)
        work = jnp.where(rio == ii, BIG, work)
        rows.append(m)
    rows += [jnp.full((1, LANES), BIG, jnp.float32)] * 3
    t5c_ref[...] = jnp.concatenate(rows, axis=0)

    @pl.when(i == NBLK - 1)
    def _finish():
        r = [t5i_ref[k:k + 1, :] for k in range(OTA_K)]
        s = ((r[0] + r[4]) + r[2]) + (r[1] + r[3])
        dkm1 = jnp.clip(s.astype(jnp.int32), 1, None) - 1
        kth = jnp.zeros((1, LANES), jnp.float32)
        for k in range(OTA_K):
            kth = kth + jnp.where(dkm1 == k, t5c_ref[k:k + 1, :], 0.0)
        kth100 = kth[:, :NUM_GT]

        lane100 = jax.lax.broadcasted_iota(jnp.int32, (1, NUM_GT), 1)
        rio_q = jax.lax.broadcasted_iota(jnp.int32, (QBLK, 1), 0)

        def body(k, carry):
            run_val, run_idx, macc = carry
            cblk = cost_ref[pl.ds(k * QBLK, QBLK), :]
            match = cblk <= kth100
            cnt = jnp.sum(match.astype(jnp.int32), axis=1, keepdims=True)
            fj = jnp.min(jnp.where(match, lane100, NUM_GT), axis=1,
                         keepdims=True)
            cmin = jnp.min(cblk, axis=1, keepdims=True)
            mg = jnp.min(jnp.where(cblk == cmin, lane100, NUM_GT), axis=1,
                         keepdims=True)
            mgt = jnp.where(cnt > 1, mg, jnp.where(cnt == 1, fj, -1))
            macc = jnp.where(lane == k, mgt, macc)
            bmin = jnp.min(cblk, axis=0, keepdims=True)
            bidx = jnp.min(jnp.where(cblk == bmin, rio_q + k * QBLK, NUM_Q),
                           axis=0, keepdims=True)
            upd = bmin < run_val
            return (jnp.where(upd, bmin, run_val),
                    jnp.where(upd, bidx, run_idx), macc)

        _, run_idx, macc = jax.lax.fori_loop(
            0, NBLK, body, (jnp.full((1, NUM_GT), BIG, jnp.float32),
                            jnp.zeros((1, NUM_GT), jnp.int32),
                            jnp.zeros((QBLK, LANES), jnp.int32)))
        mgt_ref[...] = macc
        qidx_ref[...] = run_idx


@jax.jit
def kernel(pred_logits, pred_boxes, tgt_boxes, tgt_labels):
    logits = pred_logits[0]
    boxes = pred_boxes[0]
    prob = jax.nn.sigmoid(logits)
    logp = jnp.log(prob + 1e-8)
    log1 = jnp.log(1 - prob + 1e-8)
    tgtT = jnp.zeros((8, LANES), jnp.float32).at[:, :NUM_GT].set(tgt_boxes.T)
    labels_row = jnp.zeros((1, LANES), jnp.int32).at[0, :NUM_GT].set(
        tgt_labels.astype(jnp.int32))

    cost, mgt, qidx = pl.pallas_call(
        _matcher_kernel,
        grid=(NBLK,),
        in_specs=[
            pl.BlockSpec((QBLK, NUM_C), lambda i: (i, 0)),
            pl.BlockSpec((QBLK, NUM_C), lambda i: (i, 0)),
            pl.BlockSpec((QBLK, NUM_C), lambda i: (i, 0)),
            pl.BlockSpec((QBLK, 8), lambda i: (i, 0)),
            pl.BlockSpec((8, LANES), lambda i: (0, 0)),
            pl.BlockSpec((1, LANES), lambda i: (0, 0)),
        ],
        out_specs=[
            pl.BlockSpec((NUM_Q, NUM_GT), lambda i: (0, 0)),
            pl.BlockSpec((QBLK, LANES), lambda i: (0, 0)),
            pl.BlockSpec((1, NUM_GT), lambda i: (0, 0)),
        ],
        out_shape=[
            jax.ShapeDtypeStruct((NUM_Q, NUM_GT), jnp.float32),
            jax.ShapeDtypeStruct((QBLK, LANES), jnp.int32),
            jax.ShapeDtypeStruct((1, NUM_GT), jnp.int32),
        ],
        scratch_shapes=[
            pltpu.VMEM((8, LANES), jnp.float32),
            pltpu.VMEM((8, LANES), jnp.float32),
        ],
        compiler_params=pltpu.CompilerParams(
            dimension_semantics=("arbitrary",)),
    )(logits, logp, log1, boxes, tgtT, labels_row)

    matched_gt = mgt[:, :NBLK].T.reshape(NUM_Q)
    fg_match = matched_gt >= 0
    matched_qidx = qidx.reshape(NUM_GT)
    return (fg_match, matched_gt, matched_qidx, cost)
